# baseline (speedup 1.0000x reference)
import functools

import jax
import jax.numpy as jnp
from jax import lax
from jax.experimental import pallas as pl
from jax.experimental.pallas import tpu as pltpu
from jax.experimental.pallas import tpu_sc as plsc

_WIN = 144
_RB = 16
_TB = 3584


def _conv3_body(xc_ref, xn_ref, wm_ref, bm_ref, mu_ref, xe_ref, code_ref):
    win = jnp.concatenate([xc_ref[0], xn_ref[0][:2]], axis=0)
    acc = jnp.zeros((_TB, 64), jnp.float32)
    for dy in range(3):
        for dx in range(3):
            xs = win[dy:dy + _RB, dx:dx + 224, :].reshape(_TB, 256)
            acc = acc + lax.dot_general(
                xs, wm_ref[dy * 3 + dx], (((1,), (0,)), ((), ())),
                preferred_element_type=jnp.float32)
    xe = acc + bm_ref[0]
    xe_ref[0] = jnp.concatenate([xe, jnp.zeros((_TB, 64), jnp.float32)], axis=1)
    nrm = jnp.sqrt(jnp.sum(xe * xe, axis=-1, keepdims=True))
    xn = xe / jnp.maximum(nrm, 1e-12)
    d = lax.dot_general(xn, mu_ref[...], (((1,), (1,)), ((), ())),
                        preferred_element_type=jnp.float32)
    code_ref[0, 0] = jnp.argmax(d, axis=-1).astype(jnp.int32)


def _conv3_codes(xp, wm, bm, mu, N, H, W):
    nb = H // _RB
    xe, codes = pl.pallas_call(
        _conv3_body,
        grid=(N, nb),
        in_specs=[
            pl.BlockSpec((1, _RB, W + 2, 256), lambda n, i: (n, i, 0, 0)),
            pl.BlockSpec((1, _RB, W + 2, 256), lambda n, i: (n, i + 1, 0, 0)),
            pl.BlockSpec((9, 256, 64), lambda n, i: (0, 0, 0)),
            pl.BlockSpec((1, 64), lambda n, i: (0, 0)),
            pl.BlockSpec((128, 64), lambda n, i: (0, 0)),
        ],
        out_specs=[
            pl.BlockSpec((1, _TB, 128), lambda n, i: (n, i, 0)),
            pl.BlockSpec((1, 1, _TB), lambda n, i: (n * nb + i, 0, 0)),
        ],
        out_shape=[
            jax.ShapeDtypeStruct((N, H * W, 128), jnp.float32),
            jax.ShapeDtypeStruct((N * nb, 1, _TB), jnp.int32),
        ],
    )(xp, xp, wm, bm, mu)
    return xe, codes.reshape(N, H * W)


def _conv1_body(x_ref, w_ref, b_ref, y_ref):
    y_ref[0] = lax.dot_general(
        x_ref[0], w_ref[...], (((0,), (1,)), ((), ())),
        preferred_element_type=jnp.float32) + b_ref[0]


def _conv1(x_tok, w1, b, N, L):
    nb = L // _TB
    return pl.pallas_call(
        _conv1_body,
        grid=(N, nb),
        in_specs=[
            pl.BlockSpec((1, 256, _TB), lambda n, i: (n, 0, i)),
            pl.BlockSpec((256, 256), lambda n, i: (0, 0)),
            pl.BlockSpec((1, 256), lambda n, i: (0, 0)),
        ],
        out_specs=pl.BlockSpec((1, _TB, 256), lambda n, i: (n, i, 0)),
        out_shape=jax.ShapeDtypeStruct((N, L, 256), jnp.float32),
    )(x_tok, w1, b)


def _sc_gather(tab, idx, ch):
    B = idx.shape[0]
    D = tab.shape[1]
    info = plsc.get_sparse_core_info()
    nc, ns = info.num_cores, info.num_subcores
    nw = nc * ns
    rpw = B // nw
    iters = rpw // ch
    mesh = plsc.VectorSubcoreMesh(core_axis_name="c", subcore_axis_name="s")

    @functools.partial(
        pl.kernel, mesh=mesh,
        out_type=jax.ShapeDtypeStruct((B, D), jnp.float32),
        scratch_types=[
            pltpu.VMEM((ch,), jnp.int32),
            pltpu.VMEM((ch,), jnp.int32),
            pltpu.VMEM((ch, D), jnp.float32),
            pltpu.VMEM((ch, D), jnp.float32),
            pltpu.SemaphoreType.DMA,
            pltpu.SemaphoreType.DMA,
        ],
    )
    def k(tab_hbm, idx_hbm, out_hbm, idx0, idx1, buf0, buf1, sem0, sem1):
        wid = lax.axis_index("s") * nc + lax.axis_index("c")
        base = wid * rpw
        idxs, bufs, sems = (idx0, idx1), (buf0, buf1), (sem0, sem1)
        pend = {}
        for c in range(iters):
            b = c % 2
            off = pl.multiple_of(base + c * ch, 8)
            pltpu.sync_copy(idx_hbm.at[pl.ds(off, ch)], idxs[b])
            pend[c] = pltpu.async_copy(tab_hbm.at[idxs[b]], bufs[b], sems[b])
            if c >= 1:
                pend[c - 1].wait()
                poff = pl.multiple_of(base + (c - 1) * ch, 8)
                pltpu.sync_copy(bufs[1 - b], out_hbm.at[pl.ds(poff, ch)])
        pend[iters - 1].wait()
        loff = pl.multiple_of(base + (iters - 1) * ch, 8)
        pltpu.sync_copy(bufs[(iters - 1) % 2], out_hbm.at[pl.ds(loff, ch)])

    return k(tab, idx)


def _attn_body(xc_ref, xp_ref, xn_ref, yc_ref, yp_ref, yn_ref, o_ref):
    xc = xc_ref[0, 0][:, :64]
    keys = jnp.concatenate(
        [xc_ref[0, 0][:, :64], xp_ref[0, 0][:, :64], xn_ref[0, 0][:, :64]],
        axis=0)
    knrm = jnp.sqrt(jnp.sum(keys * keys, axis=-1, keepdims=True))
    keys = keys / jnp.maximum(knrm, 5e-5)
    raw = lax.dot_general(xc.astype(jnp.bfloat16), keys.astype(jnp.bfloat16),
                          (((1,), (1,)), ((), ())),
                          preferred_element_type=jnp.float32)
    m = jnp.max(raw, axis=-1, keepdims=True)
    p = jnp.exp(raw - m)
    s = jnp.sum(p, axis=-1, keepdims=True)
    yk = jnp.concatenate([yc_ref[0, 0], yp_ref[0, 0], yn_ref[0, 0]], axis=0)
    o = lax.dot_general(p.astype(jnp.bfloat16), yk.astype(jnp.bfloat16),
                        (((1,), (0,)), ((), ())),
                        preferred_element_type=jnp.float32)
    o_ref[0, 0] = o / s


def _attention(x_s, y_s, N, nwin):
    xspec = lambda f: pl.BlockSpec((1, 1, _WIN, 128), f)
    yspec = lambda f: pl.BlockSpec((1, 1, _WIN, 256), f)
    cen = lambda n, w: (n, w, 0, 0)
    prv = lambda n, w: (n, (w + nwin - 1) % nwin, 0, 0)
    nxt = lambda n, w: (n, (w + 1) % nwin, 0, 0)
    return pl.pallas_call(
        _attn_body,
        grid=(N, nwin),
        in_specs=[xspec(cen), xspec(prv), xspec(nxt),
                  yspec(cen), yspec(prv), yspec(nxt)],
        out_specs=pl.BlockSpec((1, 1, _WIN, 256), cen),
        out_shape=jax.ShapeDtypeStruct((N, nwin, _WIN, 256), jnp.float32),
    )(x_s, x_s, x_s, y_s, y_s, y_s)


def _final_body(r_ref, x_ref, o_ref):
    o_ref[0] = jnp.transpose(r_ref[0], (1, 0)) * 0.1 + x_ref[0]


def _final(ret_u, x_tok, N, L):
    nb = L // _TB
    return pl.pallas_call(
        _final_body,
        grid=(N, nb),
        in_specs=[
            pl.BlockSpec((1, _TB, 256), lambda n, i: (n, i, 0)),
            pl.BlockSpec((1, 256, _TB), lambda n, i: (n, 0, i)),
        ],
        out_specs=pl.BlockSpec((1, 256, _TB), lambda n, i: (n, 0, i)),
        out_shape=jax.ShapeDtypeStruct((N, 256, L), jnp.float32),
    )(ret_u, x_tok)


def kernel(input_x, w_match, b_match, w_asm, b_asm, means):
    N, C, H, W = input_x.shape
    L = H * W
    rows_pad = (H // _RB + 1) * _RB - H - 1
    xp = jnp.pad(input_x.transpose(0, 2, 3, 1),
                 ((0, 0), (1, rows_pad), (1, 1), (0, 0)))
    wm = w_match.transpose(2, 3, 1, 0).reshape(9, C, 64)
    bm = b_match.reshape(1, 64)
    mu = means[0]
    x_tok = input_x.reshape(N, C, L)

    xe, codes = _conv3_codes(xp, wm, bm, mu, N, H, W)
    ye = _conv1(x_tok, w_asm.reshape(C, C), b_asm.reshape(1, C), N, L)

    indices = jnp.argsort(codes, axis=-1).astype(jnp.int32)
    bidx = jnp.arange(N, dtype=jnp.int32)[:, None]
    ranks = jnp.broadcast_to(jnp.arange(L, dtype=jnp.int32), (N, L))
    undo = jnp.zeros((N, L), jnp.int32).at[bidx, indices].set(ranks)
    pad = (_WIN - L % _WIN) % _WIN
    Lp = L + pad
    nwin = Lp // _WIN
    idx_pad = jnp.concatenate([indices, indices[:, L - pad:]], axis=1)
    offs = (jnp.arange(N, dtype=jnp.int32) * L)[:, None]
    flat_idx = (idx_pad + offs).reshape(-1)
    BP = ((N * Lp + 10239) // 10240) * 10240
    flat_idx = jnp.pad(flat_idx, (0, BP - N * Lp))

    xs_f = _sc_gather(xe.reshape(N * L, 128), flat_idx, 320)
    ys_f = _sc_gather(ye.reshape(N * L, 256), flat_idx, 160)
    x_s = xs_f[:N * Lp].reshape(N, nwin, _WIN, 128)
    y_s = ys_f[:N * Lp].reshape(N, nwin, _WIN, 256)

    ret = _attention(x_s, y_s, N, nwin)

    undo_flat = (undo + (jnp.arange(N, dtype=jnp.int32) * Lp)[:, None]).reshape(-1)
    BU = ((N * L + 10239) // 10240) * 10240
    undo_flat = jnp.pad(undo_flat, (0, BU - N * L))
    ret_u = _sc_gather(ret.reshape(N * Lp, 256), undo_flat, 160)
    ret_u = ret_u[:N * L].reshape(N, L, 256)

    out = _final(ret_u, x_tok, N, L)
    return out.reshape(N, C, H, W)


# speedup vs baseline: 1.4050x; 1.4050x over previous
import functools

import jax
import jax.numpy as jnp
from jax import lax
from jax.experimental import pallas as pl
from jax.experimental.pallas import tpu as pltpu
from jax.experimental.pallas import tpu_sc as plsc

_WIN = 144
_RB = 16
_TB = 3584


def _conv3_body(xc_ref, xn_ref, wm_ref, bm_ref, mu_ref, xe_ref, code_ref):
    win = jnp.concatenate([xc_ref[0], xn_ref[0][:2]], axis=0)
    acc = jnp.zeros((_TB, 64), jnp.float32)
    for dy in range(3):
        for dx in range(3):
            xs = win[dy:dy + _RB, dx:dx + 224, :].reshape(_TB, 256)
            acc = acc + lax.dot_general(
                xs, wm_ref[dy * 3 + dx], (((1,), (0,)), ((), ())),
                preferred_element_type=jnp.float32)
    xe = acc + bm_ref[0]
    xe_ref[0] = jnp.concatenate([xe, jnp.zeros((_TB, 64), jnp.float32)], axis=1)
    nrm = jnp.sqrt(jnp.sum(xe * xe, axis=-1, keepdims=True))
    xn = xe / jnp.maximum(nrm, 1e-12)
    d = lax.dot_general(xn, mu_ref[...], (((1,), (1,)), ((), ())),
                        preferred_element_type=jnp.float32)
    code_ref[0, 0] = jnp.argmax(d, axis=-1).astype(jnp.int32)


def _conv3_codes(xp, wm, bm, mu, N, H, W):
    nb = H // _RB
    xe, codes = pl.pallas_call(
        _conv3_body,
        grid=(N, nb),
        in_specs=[
            pl.BlockSpec((1, _RB, W + 2, 256), lambda n, i: (n, i, 0, 0)),
            pl.BlockSpec((1, _RB, W + 2, 256), lambda n, i: (n, i + 1, 0, 0)),
            pl.BlockSpec((9, 256, 64), lambda n, i: (0, 0, 0)),
            pl.BlockSpec((1, 64), lambda n, i: (0, 0)),
            pl.BlockSpec((128, 64), lambda n, i: (0, 0)),
        ],
        out_specs=[
            pl.BlockSpec((1, _TB, 128), lambda n, i: (n, i, 0)),
            pl.BlockSpec((1, 1, _TB), lambda n, i: (n * nb + i, 0, 0)),
        ],
        out_shape=[
            jax.ShapeDtypeStruct((N, H * W, 128), jnp.float32),
            jax.ShapeDtypeStruct((N * nb, 1, _TB), jnp.int32),
        ],
    )(xp, xp, wm, bm, mu)
    return xe, codes.reshape(N, H * W)


def _conv1_body(x_ref, w_ref, b_ref, y_ref):
    y_ref[0] = lax.dot_general(
        x_ref[0], w_ref[...], (((0,), (1,)), ((), ())),
        preferred_element_type=jnp.float32) + b_ref[0]


def _conv1(x_tok, w1, b, N, L):
    nb = L // _TB
    return pl.pallas_call(
        _conv1_body,
        grid=(N, nb),
        in_specs=[
            pl.BlockSpec((1, 256, _TB), lambda n, i: (n, 0, i)),
            pl.BlockSpec((256, 256), lambda n, i: (0, 0)),
            pl.BlockSpec((1, 256), lambda n, i: (0, 0)),
        ],
        out_specs=pl.BlockSpec((1, _TB, 256), lambda n, i: (n, i, 0)),
        out_shape=jax.ShapeDtypeStruct((N, L, 256), jnp.float32),
    )(x_tok, w1, b)


def _sc_gather(tab, idx, ch):
    B = idx.shape[0]
    D = tab.shape[1]
    info = plsc.get_sparse_core_info()
    nc, ns = info.num_cores, info.num_subcores
    nw = nc * ns
    rpw = B // nw
    iters = rpw // ch
    mesh = plsc.VectorSubcoreMesh(core_axis_name="c", subcore_axis_name="s")

    @functools.partial(
        pl.kernel, mesh=mesh,
        out_type=jax.ShapeDtypeStruct((B, D), jnp.float32),
        scratch_types=[
            pltpu.VMEM((ch,), jnp.int32),
            pltpu.VMEM((ch,), jnp.int32),
            pltpu.VMEM((ch, D), jnp.float32),
            pltpu.VMEM((ch, D), jnp.float32),
            pltpu.SemaphoreType.DMA,
            pltpu.SemaphoreType.DMA,
        ],
    )
    def k(tab_hbm, idx_hbm, out_hbm, idx0, idx1, buf0, buf1, sem0, sem1):
        wid = lax.axis_index("s") * nc + lax.axis_index("c")
        base = wid * rpw
        idxs, bufs, sems = (idx0, idx1), (buf0, buf1), (sem0, sem1)
        pend = {}
        for c in range(iters):
            b = c % 2
            off = pl.multiple_of(base + c * ch, 8)
            pltpu.sync_copy(idx_hbm.at[pl.ds(off, ch)], idxs[b])
            pend[c] = pltpu.async_copy(tab_hbm.at[idxs[b]], bufs[b], sems[b])
            if c >= 1:
                pend[c - 1].wait()
                poff = pl.multiple_of(base + (c - 1) * ch, 8)
                pltpu.sync_copy(bufs[1 - b], out_hbm.at[pl.ds(poff, ch)])
        pend[iters - 1].wait()
        loff = pl.multiple_of(base + (iters - 1) * ch, 8)
        pltpu.sync_copy(bufs[(iters - 1) % 2], out_hbm.at[pl.ds(loff, ch)])

    return k(tab, idx)


def _sc_scatter(src, dest_idx, out_rows, ch):
    B = src.shape[0]
    D = src.shape[1]
    info = plsc.get_sparse_core_info()
    nc, ns = info.num_cores, info.num_subcores
    nw = nc * ns
    rpw = B // nw
    iters = rpw // ch
    mesh = plsc.VectorSubcoreMesh(core_axis_name="c", subcore_axis_name="s")

    @functools.partial(
        pl.kernel, mesh=mesh,
        out_type=jax.ShapeDtypeStruct((out_rows, D), jnp.float32),
        scratch_types=[
            pltpu.VMEM((ch,), jnp.int32),
            pltpu.VMEM((ch,), jnp.int32),
            pltpu.VMEM((ch, D), jnp.float32),
            pltpu.VMEM((ch, D), jnp.float32),
            pltpu.SemaphoreType.DMA,
            pltpu.SemaphoreType.DMA,
        ],
    )
    def k(src_hbm, idx_hbm, out_hbm, idx0, idx1, buf0, buf1, sem0, sem1):
        wid = lax.axis_index("s") * nc + lax.axis_index("c")
        base = wid * rpw
        idxs, bufs, sems = (idx0, idx1), (buf0, buf1), (sem0, sem1)
        pend = {}
        for c in range(iters):
            b = c % 2
            if c >= 2:
                pend[c - 2].wait()
            off = pl.multiple_of(base + c * ch, 8)
            pltpu.sync_copy(idx_hbm.at[pl.ds(off, ch)], idxs[b])
            pltpu.sync_copy(src_hbm.at[pl.ds(off, ch)], bufs[b])
            pend[c] = pltpu.async_copy(bufs[b], out_hbm.at[idxs[b]], sems[b])
        for c in (iters - 2, iters - 1):
            pend[c].wait()

    return k(src, dest_idx)


_WB = 4


def _attn_body(xc_ref, xp_ref, xn_ref, yc_ref, yp_ref, yn_ref, o_ref):
    for w in range(_WB):
        xq = xc_ref[0, w][:, :64]
        prev = xc_ref[0, w - 1][:, :64] if w > 0 else xp_ref[0, 0][:, :64]
        nxt = xc_ref[0, w + 1][:, :64] if w < _WB - 1 else xn_ref[0, 0][:, :64]
        keys = jnp.concatenate([xq, prev, nxt], axis=0)
        knrm = jnp.sqrt(jnp.sum(keys * keys, axis=-1, keepdims=True))
        keys = keys / jnp.maximum(knrm, 5e-5)
        raw = lax.dot_general(xq.astype(jnp.bfloat16),
                              keys.astype(jnp.bfloat16),
                              (((1,), (1,)), ((), ())),
                              preferred_element_type=jnp.float32)
        m = jnp.max(raw, axis=-1, keepdims=True)
        p = jnp.exp(raw - m)
        s = jnp.sum(p, axis=-1, keepdims=True)
        yprev = yc_ref[0, w - 1] if w > 0 else yp_ref[0, 0]
        ynxt = yc_ref[0, w + 1] if w < _WB - 1 else yn_ref[0, 0]
        yk = jnp.concatenate([yc_ref[0, w], yprev, ynxt], axis=0)
        o = lax.dot_general(p.astype(jnp.bfloat16), yk.astype(jnp.bfloat16),
                            (((1,), (0,)), ((), ())),
                            preferred_element_type=jnp.float32)
        o_ref[0, w] = o / s


def _attention(x_s, y_s, N, nwin):
    ng = nwin // _WB
    cen = lambda n, g: (n, g, 0, 0)
    prv = lambda n, g: (n, (_WB * g + nwin - 1) % nwin, 0, 0)
    nxt = lambda n, g: (n, (_WB * g + _WB) % nwin, 0, 0)
    return pl.pallas_call(
        _attn_body,
        grid=(N, ng),
        in_specs=[pl.BlockSpec((1, _WB, _WIN, 128), cen),
                  pl.BlockSpec((1, 1, _WIN, 128), prv),
                  pl.BlockSpec((1, 1, _WIN, 128), nxt),
                  pl.BlockSpec((1, _WB, _WIN, 256), cen),
                  pl.BlockSpec((1, 1, _WIN, 256), prv),
                  pl.BlockSpec((1, 1, _WIN, 256), nxt)],
        out_specs=pl.BlockSpec((1, _WB, _WIN, 256), cen),
        out_shape=jax.ShapeDtypeStruct((N, nwin, _WIN, 256), jnp.float32),
    )(x_s, x_s, x_s, y_s, y_s, y_s)


def _final_body(r_ref, x_ref, o_ref):
    o_ref[0] = jnp.transpose(r_ref[0], (1, 0)) * 0.1 + x_ref[0]


def _final(ret_u, x_tok, N, L):
    nb = L // _TB
    return pl.pallas_call(
        _final_body,
        grid=(N, nb),
        in_specs=[
            pl.BlockSpec((1, _TB, 256), lambda n, i: (n, i, 0)),
            pl.BlockSpec((1, 256, _TB), lambda n, i: (n, 0, i)),
        ],
        out_specs=pl.BlockSpec((1, 256, _TB), lambda n, i: (n, 0, i)),
        out_shape=jax.ShapeDtypeStruct((N, 256, L), jnp.float32),
    )(ret_u, x_tok)


def kernel(input_x, w_match, b_match, w_asm, b_asm, means):
    N, C, H, W = input_x.shape
    L = H * W
    rows_pad = (H // _RB + 1) * _RB - H - 1
    xp = jnp.pad(input_x.transpose(0, 2, 3, 1),
                 ((0, 0), (1, rows_pad), (1, 1), (0, 0)))
    wm = w_match.transpose(2, 3, 1, 0).reshape(9, C, 64)
    bm = b_match.reshape(1, 64)
    mu = means[0]
    x_tok = input_x.reshape(N, C, L)

    xe, codes = _conv3_codes(xp, wm, bm, mu, N, H, W)
    ye = _conv1(x_tok, w_asm.reshape(C, C), b_asm.reshape(1, C), N, L)

    indices = jnp.argsort(codes, axis=-1).astype(jnp.int32)
    pad = (_WIN - L % _WIN) % _WIN
    nwin0 = (L + pad) // _WIN
    nwin = ((nwin0 + _WB - 1) // _WB) * _WB
    idx_pad = jnp.concatenate([indices, indices[:, L - pad:]], axis=1)
    wins = idx_pad.reshape(N, nwin0, _WIN)
    extra = [wins[:, :1]] * (nwin - nwin0 - 1) + [wins[:, nwin0 - 1:nwin0]]
    wins = jnp.concatenate([wins] + extra, axis=1)
    Lp = nwin * _WIN
    idx_ext = wins.reshape(N, Lp)
    offs = (jnp.arange(N, dtype=jnp.int32) * L)[:, None]
    flat_idx = (idx_ext + offs).reshape(-1)
    BP = ((N * Lp + 10239) // 10240) * 10240
    flat_idx = jnp.pad(flat_idx, (0, BP - N * Lp))

    xs_f = _sc_gather(xe.reshape(N * L, 128), flat_idx, 320)
    ys_f = _sc_gather(ye.reshape(N * L, 256), flat_idx, 160)
    x_s = xs_f[:N * Lp].reshape(N, nwin, _WIN, 128)
    y_s = ys_f[:N * Lp].reshape(N, nwin, _WIN, 256)

    ret = _attention(x_s, y_s, N, nwin)

    j_iota = jnp.broadcast_to(jnp.arange(Lp, dtype=jnp.int32), (N, Lp))
    dest = jnp.where(j_iota < L, idx_ext + offs, N * L)
    dest_flat = dest.reshape(-1)
    sch = 176 if (N * Lp // 32) % 176 == 0 else 8
    ret_u = _sc_scatter(ret.reshape(N * Lp, 256), dest_flat, N * L + 8, sch)
    ret_u = ret_u[:N * L].reshape(N, L, 256)

    out = _final(ret_u, x_tok, N, L)
    return out.reshape(N, C, H, W)


# speedup vs baseline: 1.4142x; 1.0065x over previous
import functools

import jax
import jax.numpy as jnp
from jax import lax
from jax.experimental import pallas as pl
from jax.experimental.pallas import tpu as pltpu
from jax.experimental.pallas import tpu_sc as plsc

_WIN = 144
_RB = 16
_TB = 3584


def _conv3_body(xc_ref, xn_ref, wm_ref, bm_ref, mu_ref, xe_ref, code_ref):
    win = jnp.concatenate([xc_ref[0], xn_ref[0][:2]], axis=0)
    acc = jnp.zeros((_TB, 64), jnp.float32)
    for dx in range(3):
        wdx = win[:, dx:dx + 224, :]
        for dy in range(3):
            xs = wdx[dy:dy + _RB].reshape(_TB, 256)
            acc = acc + lax.dot_general(
                xs, wm_ref[dy * 3 + dx], (((1,), (0,)), ((), ())),
                preferred_element_type=jnp.float32)
    xe = acc + bm_ref[0]
    xe_ref[0] = jnp.concatenate([xe, jnp.zeros((_TB, 64), jnp.float32)], axis=1)
    nrm = jnp.sqrt(jnp.sum(xe * xe, axis=-1, keepdims=True))
    xn = xe / jnp.maximum(nrm, 1e-12)
    d = lax.dot_general(xn, mu_ref[...], (((1,), (1,)), ((), ())),
                        preferred_element_type=jnp.float32)
    code_ref[0, 0] = jnp.argmax(d, axis=-1).astype(jnp.int32)


def _conv3_codes(xp, wm, bm, mu, N, H, W):
    nb = H // _RB
    xe, codes = pl.pallas_call(
        _conv3_body,
        grid=(N, nb),
        in_specs=[
            pl.BlockSpec((1, _RB, W + 2, 256), lambda n, i: (n, i, 0, 0)),
            pl.BlockSpec((1, _RB, W + 2, 256), lambda n, i: (n, i + 1, 0, 0)),
            pl.BlockSpec((9, 256, 64), lambda n, i: (0, 0, 0)),
            pl.BlockSpec((1, 64), lambda n, i: (0, 0)),
            pl.BlockSpec((128, 64), lambda n, i: (0, 0)),
        ],
        out_specs=[
            pl.BlockSpec((1, _TB, 128), lambda n, i: (n, i, 0)),
            pl.BlockSpec((1, 1, _TB), lambda n, i: (n * nb + i, 0, 0)),
        ],
        out_shape=[
            jax.ShapeDtypeStruct((N, H * W, 128), jnp.float32),
            jax.ShapeDtypeStruct((N * nb, 1, _TB), jnp.int32),
        ],
    )(xp, xp, wm, bm, mu)
    return xe, codes.reshape(N, H * W)


def _conv1_body(x_ref, w_ref, b_ref, y_ref):
    y_ref[0] = lax.dot_general(
        x_ref[0], w_ref[...], (((0,), (1,)), ((), ())),
        preferred_element_type=jnp.float32) + b_ref[0]


def _conv1(x_tok, w1, b, N, L):
    nb = L // _TB
    return pl.pallas_call(
        _conv1_body,
        grid=(N, nb),
        in_specs=[
            pl.BlockSpec((1, 256, _TB), lambda n, i: (n, 0, i)),
            pl.BlockSpec((256, 256), lambda n, i: (0, 0)),
            pl.BlockSpec((1, 256), lambda n, i: (0, 0)),
        ],
        out_specs=pl.BlockSpec((1, _TB, 256), lambda n, i: (n, i, 0)),
        out_shape=jax.ShapeDtypeStruct((N, L, 256), jnp.float32),
    )(x_tok, w1, b)


def _sc_gather(tab, idx, ch):
    B = idx.shape[0]
    D = tab.shape[1]
    info = plsc.get_sparse_core_info()
    nc, ns = info.num_cores, info.num_subcores
    nw = nc * ns
    rpw = B // nw
    iters = rpw // ch
    mesh = plsc.VectorSubcoreMesh(core_axis_name="c", subcore_axis_name="s")

    @functools.partial(
        pl.kernel, mesh=mesh,
        out_type=jax.ShapeDtypeStruct((B, D), jnp.float32),
        scratch_types=[
            pltpu.VMEM((ch,), jnp.int32),
            pltpu.VMEM((ch,), jnp.int32),
            pltpu.VMEM((ch, D), jnp.float32),
            pltpu.VMEM((ch, D), jnp.float32),
            pltpu.SemaphoreType.DMA,
            pltpu.SemaphoreType.DMA,
        ],
    )
    def k(tab_hbm, idx_hbm, out_hbm, idx0, idx1, buf0, buf1, sem0, sem1):
        wid = lax.axis_index("s") * nc + lax.axis_index("c")
        base = wid * rpw
        idxs, bufs, sems = (idx0, idx1), (buf0, buf1), (sem0, sem1)
        pend = {}
        for c in range(iters):
            b = c % 2
            off = pl.multiple_of(base + c * ch, 8)
            pltpu.sync_copy(idx_hbm.at[pl.ds(off, ch)], idxs[b])
            pend[c] = pltpu.async_copy(tab_hbm.at[idxs[b]], bufs[b], sems[b])
            if c >= 1:
                pend[c - 1].wait()
                poff = pl.multiple_of(base + (c - 1) * ch, 8)
                pltpu.sync_copy(bufs[1 - b], out_hbm.at[pl.ds(poff, ch)])
        pend[iters - 1].wait()
        loff = pl.multiple_of(base + (iters - 1) * ch, 8)
        pltpu.sync_copy(bufs[(iters - 1) % 2], out_hbm.at[pl.ds(loff, ch)])

    return k(tab, idx)


def _sc_scatter(src, dest_idx, out_rows, ch):
    B = src.shape[0]
    D = src.shape[1]
    info = plsc.get_sparse_core_info()
    nc, ns = info.num_cores, info.num_subcores
    nw = nc * ns
    rpw = B // nw
    iters = rpw // ch
    mesh = plsc.VectorSubcoreMesh(core_axis_name="c", subcore_axis_name="s")

    @functools.partial(
        pl.kernel, mesh=mesh,
        out_type=jax.ShapeDtypeStruct((out_rows, D), jnp.float32),
        scratch_types=[
            pltpu.VMEM((ch,), jnp.int32),
            pltpu.VMEM((ch,), jnp.int32),
            pltpu.VMEM((ch, D), jnp.float32),
            pltpu.VMEM((ch, D), jnp.float32),
            pltpu.SemaphoreType.DMA,
            pltpu.SemaphoreType.DMA,
        ],
    )
    def k(src_hbm, idx_hbm, out_hbm, idx0, idx1, buf0, buf1, sem0, sem1):
        wid = lax.axis_index("s") * nc + lax.axis_index("c")
        base = wid * rpw
        idxs, bufs, sems = (idx0, idx1), (buf0, buf1), (sem0, sem1)
        pend = {}
        for c in range(iters):
            b = c % 2
            if c >= 2:
                pend[c - 2].wait()
            off = pl.multiple_of(base + c * ch, 8)
            pltpu.sync_copy(idx_hbm.at[pl.ds(off, ch)], idxs[b])
            pltpu.sync_copy(src_hbm.at[pl.ds(off, ch)], bufs[b])
            pend[c] = pltpu.async_copy(bufs[b], out_hbm.at[idxs[b]], sems[b])
        for c in (iters - 2, iters - 1):
            pend[c].wait()

    return k(src, dest_idx)


_WB = 8


def _attn_body(xc_ref, xp_ref, xn_ref, yc_ref, yp_ref, yn_ref, o_ref):
    for w in range(_WB):
        xq = xc_ref[0, w][:, :64]
        prev = xc_ref[0, w - 1][:, :64] if w > 0 else xp_ref[0, 0][:, :64]
        nxt = xc_ref[0, w + 1][:, :64] if w < _WB - 1 else xn_ref[0, 0][:, :64]
        keys = jnp.concatenate([xq, prev, nxt], axis=0)
        knrm = jnp.sqrt(jnp.sum(keys * keys, axis=-1, keepdims=True))
        keys = keys / jnp.maximum(knrm, 5e-5)
        raw = lax.dot_general(xq.astype(jnp.bfloat16),
                              keys.astype(jnp.bfloat16),
                              (((1,), (1,)), ((), ())),
                              preferred_element_type=jnp.float32)
        m = jnp.max(raw, axis=-1, keepdims=True)
        p = jnp.exp(raw - m)
        s = jnp.sum(p, axis=-1, keepdims=True)
        yprev = yc_ref[0, w - 1] if w > 0 else yp_ref[0, 0]
        ynxt = yc_ref[0, w + 1] if w < _WB - 1 else yn_ref[0, 0]
        yk = jnp.concatenate([yc_ref[0, w], yprev, ynxt], axis=0)
        o = lax.dot_general(p.astype(jnp.bfloat16), yk.astype(jnp.bfloat16),
                            (((1,), (0,)), ((), ())),
                            preferred_element_type=jnp.float32)
        o_ref[0, w] = o / s


def _attention(x_s, y_s, N, nwin):
    ng = nwin // _WB
    cen = lambda n, g: (n, g, 0, 0)
    prv = lambda n, g: (n, (_WB * g + nwin - 1) % nwin, 0, 0)
    nxt = lambda n, g: (n, (_WB * g + _WB) % nwin, 0, 0)
    return pl.pallas_call(
        _attn_body,
        grid=(N, ng),
        in_specs=[pl.BlockSpec((1, _WB, _WIN, 128), cen),
                  pl.BlockSpec((1, 1, _WIN, 128), prv),
                  pl.BlockSpec((1, 1, _WIN, 128), nxt),
                  pl.BlockSpec((1, _WB, _WIN, 256), cen),
                  pl.BlockSpec((1, 1, _WIN, 256), prv),
                  pl.BlockSpec((1, 1, _WIN, 256), nxt)],
        out_specs=pl.BlockSpec((1, _WB, _WIN, 256), cen),
        out_shape=jax.ShapeDtypeStruct((N, nwin, _WIN, 256), jnp.float32),
    )(x_s, x_s, x_s, y_s, y_s, y_s)


def _final_body(r_ref, x_ref, o_ref):
    o_ref[0] = jnp.transpose(r_ref[0], (1, 0)) * 0.1 + x_ref[0]


def _final(ret_u, x_tok, N, L):
    nb = L // _TB
    return pl.pallas_call(
        _final_body,
        grid=(N, nb),
        in_specs=[
            pl.BlockSpec((1, _TB, 256), lambda n, i: (n, i, 0)),
            pl.BlockSpec((1, 256, _TB), lambda n, i: (n, 0, i)),
        ],
        out_specs=pl.BlockSpec((1, 256, _TB), lambda n, i: (n, 0, i)),
        out_shape=jax.ShapeDtypeStruct((N, 256, L), jnp.float32),
    )(ret_u, x_tok)


def kernel(input_x, w_match, b_match, w_asm, b_asm, means):
    N, C, H, W = input_x.shape
    L = H * W
    rows_pad = (H // _RB + 1) * _RB - H - 1
    xp = jnp.pad(input_x.transpose(0, 2, 3, 1),
                 ((0, 0), (1, rows_pad), (1, 1), (0, 0)))
    wm = w_match.transpose(2, 3, 1, 0).reshape(9, C, 64)
    bm = b_match.reshape(1, 64)
    mu = means[0]
    x_tok = input_x.reshape(N, C, L)

    xe, codes = _conv3_codes(xp, wm, bm, mu, N, H, W)
    ye = _conv1(x_tok, w_asm.reshape(C, C), b_asm.reshape(1, C), N, L)

    indices = jnp.argsort(codes, axis=-1).astype(jnp.int32)
    pad = (_WIN - L % _WIN) % _WIN
    nwin0 = (L + pad) // _WIN
    nwin = ((nwin0 + _WB - 1) // _WB) * _WB
    idx_pad = jnp.concatenate([indices, indices[:, L - pad:]], axis=1)
    wins = idx_pad.reshape(N, nwin0, _WIN)
    extra = [wins[:, :1]] * (nwin - nwin0 - 1) + [wins[:, nwin0 - 1:nwin0]]
    wins = jnp.concatenate([wins] + extra, axis=1)
    Lp = nwin * _WIN
    idx_ext = wins.reshape(N, Lp)
    offs = (jnp.arange(N, dtype=jnp.int32) * L)[:, None]
    flat_idx = (idx_ext + offs).reshape(-1)
    BP = ((N * Lp + 10239) // 10240) * 10240
    flat_idx = jnp.pad(flat_idx, (0, BP - N * Lp))

    xs_f = _sc_gather(xe.reshape(N * L, 128), flat_idx, 320)
    ys_f = _sc_gather(ye.reshape(N * L, 256), flat_idx, 160)
    x_s = xs_f[:N * Lp].reshape(N, nwin, _WIN, 128)
    y_s = ys_f[:N * Lp].reshape(N, nwin, _WIN, 256)

    ret = _attention(x_s, y_s, N, nwin)

    j_iota = jnp.broadcast_to(jnp.arange(Lp, dtype=jnp.int32), (N, Lp))
    dest = jnp.where(j_iota < L, idx_ext + offs, N * L)
    dest_flat = dest.reshape(-1)
    sch = 176 if (N * Lp // 32) % 176 == 0 else 8
    ret_u = _sc_scatter(ret.reshape(N * Lp, 256), dest_flat, N * L + 8, sch)
    ret_u = ret_u[:N * L].reshape(N, L, 256)

    out = _final(ret_u, x_tok, N, L)
    return out.reshape(N, C, H, W)


# speedup vs baseline: 1.6664x; 1.1783x over previous
import functools

import jax
import jax.numpy as jnp
from jax import lax
from jax.experimental import pallas as pl
from jax.experimental.pallas import tpu as pltpu
from jax.experimental.pallas import tpu_sc as plsc

_WIN = 144
_RB = 16
_TB = 3584


def _conv3_body(xc_ref, xn_ref, wm_ref, bm_ref, mu_ref, xe_ref, code_ref):
    win = jnp.concatenate([xc_ref[0], xn_ref[0][:2]], axis=0)
    acc = jnp.zeros((_TB, 64), jnp.float32)
    for dx in range(3):
        wdx = win[:, dx:dx + 224, :]
        for dy in range(3):
            xs = wdx[dy:dy + _RB].reshape(_TB, 256)
            acc = acc + lax.dot_general(
                xs, wm_ref[dy * 3 + dx], (((1,), (0,)), ((), ())),
                preferred_element_type=jnp.float32)
    xe = acc + bm_ref[0]
    xe_ref[0] = jnp.concatenate([xe, jnp.zeros((_TB, 64), jnp.float32)], axis=1)
    nrm = jnp.sqrt(jnp.sum(xe * xe, axis=-1, keepdims=True))
    xn = xe / jnp.maximum(nrm, 1e-12)
    d = lax.dot_general(xn, mu_ref[...], (((1,), (1,)), ((), ())),
                        preferred_element_type=jnp.float32)
    code_ref[0, 0] = jnp.argmax(d, axis=-1).astype(jnp.int32)


def _conv3_codes(xp, wm, bm, mu, N, H, W):
    nb = H // _RB
    xe, codes = pl.pallas_call(
        _conv3_body,
        grid=(N, nb),
        in_specs=[
            pl.BlockSpec((1, _RB, W + 2, 256), lambda n, i: (n, i, 0, 0)),
            pl.BlockSpec((1, _RB, W + 2, 256), lambda n, i: (n, i + 1, 0, 0)),
            pl.BlockSpec((9, 256, 64), lambda n, i: (0, 0, 0)),
            pl.BlockSpec((1, 64), lambda n, i: (0, 0)),
            pl.BlockSpec((128, 64), lambda n, i: (0, 0)),
        ],
        out_specs=[
            pl.BlockSpec((1, _TB, 128), lambda n, i: (n, i, 0)),
            pl.BlockSpec((1, 1, _TB), lambda n, i: (n * nb + i, 0, 0)),
        ],
        out_shape=[
            jax.ShapeDtypeStruct((N, H * W, 128), jnp.float32),
            jax.ShapeDtypeStruct((N * nb, 1, _TB), jnp.int32),
        ],
    )(xp, xp, wm, bm, mu)
    return xe, codes.reshape(N, H * W)


def _conv1_body(x_ref, w_ref, b_ref, y_ref):
    y_ref[0] = lax.dot_general(
        x_ref[0], w_ref[...], (((0,), (1,)), ((), ())),
        preferred_element_type=jnp.float32) + b_ref[0]


def _conv1(x_tok, w1, b, N, L):
    nb = L // _TB
    return pl.pallas_call(
        _conv1_body,
        grid=(N, nb),
        in_specs=[
            pl.BlockSpec((1, 256, _TB), lambda n, i: (n, 0, i)),
            pl.BlockSpec((256, 256), lambda n, i: (0, 0)),
            pl.BlockSpec((1, 256), lambda n, i: (0, 0)),
        ],
        out_specs=pl.BlockSpec((1, _TB, 256), lambda n, i: (n, i, 0)),
        out_shape=jax.ShapeDtypeStruct((N, L, 256), jnp.float32),
    )(x_tok, w1, b)


def _sc_gather(tab, idx, ch):
    B = idx.shape[0]
    D = tab.shape[1]
    info = plsc.get_sparse_core_info()
    nc, ns = info.num_cores, info.num_subcores
    nw = nc * ns
    rpw = B // nw
    iters = rpw // ch
    mesh = plsc.VectorSubcoreMesh(core_axis_name="c", subcore_axis_name="s")

    @functools.partial(
        pl.kernel, mesh=mesh,
        out_type=jax.ShapeDtypeStruct((B, D), jnp.float32),
        scratch_types=[
            pltpu.VMEM((ch,), jnp.int32),
            pltpu.VMEM((ch,), jnp.int32),
            pltpu.VMEM((ch, D), jnp.float32),
            pltpu.VMEM((ch, D), jnp.float32),
            pltpu.SemaphoreType.DMA,
            pltpu.SemaphoreType.DMA,
        ],
    )
    def k(tab_hbm, idx_hbm, out_hbm, idx0, idx1, buf0, buf1, sem0, sem1):
        wid = lax.axis_index("s") * nc + lax.axis_index("c")
        base = wid * rpw
        idxs, bufs, sems = (idx0, idx1), (buf0, buf1), (sem0, sem1)
        pend = {}
        for c in range(iters):
            b = c % 2
            off = pl.multiple_of(base + c * ch, 8)
            pltpu.sync_copy(idx_hbm.at[pl.ds(off, ch)], idxs[b])
            pend[c] = pltpu.async_copy(tab_hbm.at[idxs[b]], bufs[b], sems[b])
            if c >= 1:
                pend[c - 1].wait()
                poff = pl.multiple_of(base + (c - 1) * ch, 8)
                pltpu.sync_copy(bufs[1 - b], out_hbm.at[pl.ds(poff, ch)])
        pend[iters - 1].wait()
        loff = pl.multiple_of(base + (iters - 1) * ch, 8)
        pltpu.sync_copy(bufs[(iters - 1) % 2], out_hbm.at[pl.ds(loff, ch)])

    return k(tab, idx)


def _sc_scatter(src, dest_idx, out_rows, ch):
    B = src.shape[0]
    D = src.shape[1]
    info = plsc.get_sparse_core_info()
    nc, ns = info.num_cores, info.num_subcores
    nw = nc * ns
    rpw = B // nw
    iters = rpw // ch
    mesh = plsc.VectorSubcoreMesh(core_axis_name="c", subcore_axis_name="s")

    @functools.partial(
        pl.kernel, mesh=mesh,
        out_type=jax.ShapeDtypeStruct((out_rows, D), jnp.float32),
        scratch_types=[
            pltpu.VMEM((ch,), jnp.int32),
            pltpu.VMEM((ch,), jnp.int32),
            pltpu.VMEM((ch, D), jnp.float32),
            pltpu.VMEM((ch, D), jnp.float32),
            pltpu.SemaphoreType.DMA,
            pltpu.SemaphoreType.DMA,
        ],
    )
    def k(src_hbm, idx_hbm, out_hbm, idx0, idx1, buf0, buf1, sem0, sem1):
        wid = lax.axis_index("s") * nc + lax.axis_index("c")
        base = wid * rpw
        idxs, bufs, sems = (idx0, idx1), (buf0, buf1), (sem0, sem1)
        pend = {}
        for c in range(iters):
            b = c % 2
            if c >= 2:
                pend[c - 2].wait()
            off = pl.multiple_of(base + c * ch, 8)
            pltpu.sync_copy(idx_hbm.at[pl.ds(off, ch)], idxs[b])
            pltpu.sync_copy(src_hbm.at[pl.ds(off, ch)], bufs[b])
            pend[c] = pltpu.async_copy(bufs[b], out_hbm.at[idxs[b]], sems[b])
        for c in (iters - 2, iters - 1):
            pend[c].wait()

    return k(src, dest_idx)


_WB = 8


def _attn_body(xc_ref, xp_ref, xn_ref, yc_ref, yp_ref, yn_ref, o_ref):
    for w in range(_WB):
        xq = xc_ref[0, w][:, :64]
        prev = xc_ref[0, w - 1][:, :64] if w > 0 else xp_ref[0, 0][:, :64]
        nxt = xc_ref[0, w + 1][:, :64] if w < _WB - 1 else xn_ref[0, 0][:, :64]
        keys = jnp.concatenate([xq, prev, nxt], axis=0)
        knrm = jnp.sqrt(jnp.sum(keys * keys, axis=-1, keepdims=True))
        keys = keys / jnp.maximum(knrm, 5e-5)
        raw = lax.dot_general(xq.astype(jnp.bfloat16),
                              keys.astype(jnp.bfloat16),
                              (((1,), (1,)), ((), ())),
                              preferred_element_type=jnp.float32)
        m = jnp.max(raw, axis=-1, keepdims=True)
        p = jnp.exp(raw - m)
        s = jnp.sum(p, axis=-1, keepdims=True)
        yprev = yc_ref[0, w - 1] if w > 0 else yp_ref[0, 0]
        ynxt = yc_ref[0, w + 1] if w < _WB - 1 else yn_ref[0, 0]
        yk = jnp.concatenate([yc_ref[0, w], yprev, ynxt], axis=0)
        o = lax.dot_general(p.astype(jnp.bfloat16), yk.astype(jnp.bfloat16),
                            (((1,), (0,)), ((), ())),
                            preferred_element_type=jnp.float32)
        o_ref[0, w] = o / s


def _attention(x_s, y_s, N, nwin):
    ng = nwin // _WB
    cen = lambda n, g: (n, g, 0, 0)
    prv = lambda n, g: (n, (_WB * g + nwin - 1) % nwin, 0, 0)
    nxt = lambda n, g: (n, (_WB * g + _WB) % nwin, 0, 0)
    return pl.pallas_call(
        _attn_body,
        grid=(N, ng),
        in_specs=[pl.BlockSpec((1, _WB, _WIN, 128), cen),
                  pl.BlockSpec((1, 1, _WIN, 128), prv),
                  pl.BlockSpec((1, 1, _WIN, 128), nxt),
                  pl.BlockSpec((1, _WB, _WIN, 256), cen),
                  pl.BlockSpec((1, 1, _WIN, 256), prv),
                  pl.BlockSpec((1, 1, _WIN, 256), nxt)],
        out_specs=pl.BlockSpec((1, _WB, _WIN, 256), cen),
        out_shape=jax.ShapeDtypeStruct((N, nwin, _WIN, 256), jnp.float32),
    )(x_s, x_s, x_s, y_s, y_s, y_s)


def _final_body(r_ref, x_ref, o_ref):
    o_ref[0] = jnp.transpose(r_ref[...], (1, 0)) * 0.1 + x_ref[0]


def _final(ret_u2d, x_tok, N, L):
    nb = L // _TB
    return pl.pallas_call(
        _final_body,
        grid=(N, nb),
        in_specs=[
            pl.BlockSpec((_TB, 256), lambda n, i: (n * nb + i, 0)),
            pl.BlockSpec((1, 256, _TB), lambda n, i: (n, 0, i)),
        ],
        out_specs=pl.BlockSpec((1, 256, _TB), lambda n, i: (n, 0, i)),
        out_shape=jax.ShapeDtypeStruct((N, 256, L), jnp.float32),
    )(ret_u2d, x_tok)


def kernel(input_x, w_match, b_match, w_asm, b_asm, means):
    N, C, H, W = input_x.shape
    L = H * W
    rows_pad = (H // _RB + 1) * _RB - H - 1
    xp = jnp.pad(input_x.transpose(0, 2, 3, 1),
                 ((0, 0), (1, rows_pad), (1, 1), (0, 0)))
    wm = w_match.transpose(2, 3, 1, 0).reshape(9, C, 64)
    bm = b_match.reshape(1, 64)
    mu = means[0]
    x_tok = input_x.reshape(N, C, L)

    xe, codes = _conv3_codes(xp, wm, bm, mu, N, H, W)
    ye = _conv1(x_tok, w_asm.reshape(C, C), b_asm.reshape(1, C), N, L)

    indices = jnp.argsort(codes, axis=-1).astype(jnp.int32)
    pad = (_WIN - L % _WIN) % _WIN
    nwin0 = (L + pad) // _WIN
    nwin = ((nwin0 + _WB - 1) // _WB) * _WB
    idx_pad = jnp.concatenate([indices, indices[:, L - pad:]], axis=1)
    wins = idx_pad.reshape(N, nwin0, _WIN)
    extra = [wins[:, :1]] * (nwin - nwin0 - 1) + [wins[:, nwin0 - 1:nwin0]]
    wins = jnp.concatenate([wins] + extra, axis=1)
    Lp = nwin * _WIN
    idx_ext = wins.reshape(N, Lp)
    offs = (jnp.arange(N, dtype=jnp.int32) * L)[:, None]
    flat_idx = (idx_ext + offs).reshape(-1)

    xs_f = _sc_gather(xe.reshape(N * L, 128), flat_idx, 352)
    ys_f = _sc_gather(ye.reshape(N * L, 256), flat_idx, 176)
    x_s = xs_f.reshape(N, nwin, _WIN, 128)
    y_s = ys_f.reshape(N, nwin, _WIN, 256)

    ret = _attention(x_s, y_s, N, nwin)

    j_iota = jnp.broadcast_to(jnp.arange(Lp, dtype=jnp.int32), (N, Lp))
    dest = jnp.where(j_iota < L, idx_ext + offs, N * L)
    dest_flat = dest.reshape(-1)
    outr = ((N * L + 1 + _TB - 1) // _TB) * _TB
    ret_u = _sc_scatter(ret.reshape(N * Lp, 256), dest_flat, outr, 176)

    out = _final(ret_u, x_tok, N, L)
    return out.reshape(N, C, H, W)
